# Initial kernel scaffold; baseline (speedup 1.0000x reference)
import jax
import jax.numpy as jnp
from jax.experimental import pallas as pl


def kernel(tokens, table):
    raise NotImplementedError("write your pallas kernel here")


# speedup vs baseline: 1.4925x; 1.4925x over previous
import functools

import jax
import jax.numpy as jnp
from jax import lax
from jax.experimental import pallas as pl
from jax.experimental.pallas import tpu as pltpu
from jax.experimental.pallas import tpu_sc as plsc

D = 256
NC, NS = 2, 16
NW = NC * NS
W = 128


def _sc_gather(tokens_flat, table):
    B = tokens_flat.shape[0]
    b_per_w = B // NW
    n_chunks = b_per_w // W
    mesh = plsc.VectorSubcoreMesh(core_axis_name="c", subcore_axis_name="s")

    @functools.partial(
        pl.kernel,
        mesh=mesh,
        out_type=jax.ShapeDtypeStruct((B, D), jnp.float32),
        scratch_types=[
            pltpu.VMEM((W,), jnp.int32),
            pltpu.VMEM((W, D), jnp.float32),
            pltpu.SemaphoreType.DMA,
        ],
    )
    def k(table_hbm, idx_hbm, out_hbm, idx_v, rows_v, sem):
        wid = lax.axis_index("s") * NC + lax.axis_index("c")
        base = wid * b_per_w

        @pl.loop(0, n_chunks)
        def _(c):
            off = base + c * W
            pltpu.sync_copy(idx_hbm.at[pl.ds(off, W)], idx_v)
            pltpu.async_copy(table_hbm.at[idx_v], rows_v, sem).wait()
            pltpu.sync_copy(rows_v, out_hbm.at[pl.ds(off, W)])

    return k(table, tokens_flat)


def kernel(tokens, table):
    bsz, seq = tokens.shape
    out = _sc_gather(tokens.reshape(bsz * seq), table)
    return out.reshape(bsz, seq, D)


# speedup vs baseline: 1.4959x; 1.0023x over previous
import functools

import jax
import jax.numpy as jnp
from jax import lax
from jax.experimental import pallas as pl
from jax.experimental.pallas import tpu as pltpu
from jax.experimental.pallas import tpu_sc as plsc

D = 256
NC, NS = 2, 16
NW = NC * NS
W = 128
IB = 2048
CPB = IB // W


def _sc_gather(tokens_flat, table):
    B = tokens_flat.shape[0]
    b_per_w = B // NW
    n_outer = b_per_w // IB
    mesh = plsc.VectorSubcoreMesh(core_axis_name="c", subcore_axis_name="s")

    @functools.partial(
        pl.kernel,
        mesh=mesh,
        out_type=jax.ShapeDtypeStruct((B, D), jnp.float32),
        scratch_types=[
            pltpu.VMEM((IB,), jnp.int32),
            pltpu.VMEM((W, D), jnp.float32),
            pltpu.VMEM((W, D), jnp.float32),
            pltpu.SemaphoreType.DMA,
            pltpu.SemaphoreType.DMA,
            pltpu.SemaphoreType.DMA,
            pltpu.SemaphoreType.DMA,
        ],
    )
    def k(table_hbm, idx_hbm, out_hbm, idx_v, rows0, rows1, g0, g1, w0, w1):
        rows = (rows0, rows1)
        gs = (g0, g1)
        ws = (w0, w1)
        wid = lax.axis_index("s") * NC + lax.axis_index("c")
        base = wid * b_per_w

        @pl.loop(0, n_outer)
        def _(o):
            obase = base + o * IB
            pltpu.sync_copy(idx_hbm.at[pl.ds(obase, IB)], idx_v)

            @pl.loop(0, CPB, step=2)
            def _(ci):
                handles = []
                for b in range(2):
                    c = ci + b
                    off = obase + c * W

                    @pl.when(jnp.logical_or(o > 0, ci >= 2))
                    def _():
                        pltpu.make_async_copy(
                            rows[b], out_hbm.at[pl.ds(off, W)], ws[b]
                        ).wait()

                    handles.append(
                        pltpu.async_copy(
                            table_hbm.at[idx_v.at[pl.ds(c * W, W)]],
                            rows[b],
                            gs[b],
                        )
                    )
                for b in range(2):
                    c = ci + b
                    off = obase + c * W
                    handles[b].wait()
                    pltpu.async_copy(rows[b], out_hbm.at[pl.ds(off, W)], ws[b])

        for b in range(2):
            pltpu.make_async_copy(
                rows[b], out_hbm.at[pl.ds(base, W)], ws[b]
            ).wait()

    return k(table, tokens_flat)


def kernel(tokens, table):
    bsz, seq = tokens.shape
    out = _sc_gather(tokens.reshape(bsz * seq), table)
    return out.reshape(bsz, seq, D)


# speedup vs baseline: 4.4097x; 2.9479x over previous
import functools

import jax
import jax.numpy as jnp
from jax import lax
from jax.experimental import pallas as pl
from jax.experimental.pallas import tpu as pltpu
from jax.experimental.pallas import tpu_sc as plsc

D = 256
NC, NS = 2, 16
NW = NC * NS
W = 128
IB = 2048
CPB = IB // W


def _sc_gather(tokens_flat, table_repl):
    B = tokens_flat.shape[0]
    V = table_repl.shape[1]
    b_per_w = B // NW
    n_outer = b_per_w // IB
    mesh = plsc.VectorSubcoreMesh(core_axis_name="c", subcore_axis_name="s")

    @functools.partial(
        pl.kernel,
        mesh=mesh,
        out_type=jax.ShapeDtypeStruct((B, D), jnp.float32),
        scratch_types=[
            pltpu.VMEM((IB,), jnp.int32),
            pltpu.VMEM((W, D), jnp.float32),
            pltpu.VMEM((W, D), jnp.float32),
            pltpu.SemaphoreType.DMA,
            pltpu.SemaphoreType.DMA,
            pltpu.SemaphoreType.DMA,
            pltpu.SemaphoreType.DMA,
        ],
    )
    def k(tab_hbm, idx_hbm, out_hbm, idx_v, rows0, rows1, g0, g1, w0, w1):
        rows = (rows0, rows1)
        gs = (g0, g1)
        ws = (w0, w1)
        wid = lax.axis_index("s") * NC + lax.axis_index("c")
        base = wid * b_per_w
        my_tab = tab_hbm.at[wid]

        @pl.loop(0, n_outer)
        def _(o):
            obase = base + o * IB
            pltpu.sync_copy(idx_hbm.at[pl.ds(obase, IB)], idx_v)

            @pl.loop(0, CPB, step=2)
            def _(ci):
                handles = []
                for b in range(2):
                    c = ci + b
                    off = obase + c * W

                    @pl.when(jnp.logical_or(o > 0, ci >= 2))
                    def _():
                        pltpu.make_async_copy(
                            rows[b], out_hbm.at[pl.ds(off, W)], ws[b]
                        ).wait()

                    handles.append(
                        pltpu.async_copy(
                            my_tab.at[idx_v.at[pl.ds(c * W, W)]],
                            rows[b],
                            gs[b],
                        )
                    )
                for b in range(2):
                    c = ci + b
                    off = obase + c * W
                    handles[b].wait()
                    pltpu.async_copy(rows[b], out_hbm.at[pl.ds(off, W)], ws[b])

        for b in range(2):
            pltpu.make_async_copy(
                rows[b], out_hbm.at[pl.ds(base, W)], ws[b]
            ).wait()

    return k(table_repl, tokens_flat)


def kernel(tokens, table):
    bsz, seq = tokens.shape
    table_repl = jnp.broadcast_to(table, (NW,) + table.shape)
    out = _sc_gather(tokens.reshape(bsz * seq), table_repl)
    return out.reshape(bsz, seq, D)


# speedup vs baseline: 4.4460x; 1.0082x over previous
import functools

import jax
import jax.numpy as jnp
from jax import lax
from jax.experimental import pallas as pl
from jax.experimental.pallas import tpu as pltpu
from jax.experimental.pallas import tpu_sc as plsc

D = 256
NC, NS = 2, 16
NW = NC * NS
W = 64
NB = 4
IB = 2048
CPB = IB // W


def _sc_gather(tokens_flat, table_repl):
    B = tokens_flat.shape[0]
    b_per_w = B // NW
    n_outer = b_per_w // IB
    mesh = plsc.VectorSubcoreMesh(core_axis_name="c", subcore_axis_name="s")

    @functools.partial(
        pl.kernel,
        mesh=mesh,
        out_type=jax.ShapeDtypeStruct((B, D), jnp.float32),
        scratch_types=(
            [pltpu.VMEM((IB,), jnp.int32)]
            + [pltpu.VMEM((W, D), jnp.float32)] * NB
            + [pltpu.SemaphoreType.DMA] * (2 * NB)
        ),
    )
    def k(tab_hbm, idx_hbm, out_hbm, idx_v, *bufs):
        rows = bufs[:NB]
        gs = bufs[NB:2 * NB]
        ws = bufs[2 * NB:3 * NB]
        wid = lax.axis_index("s") * NC + lax.axis_index("c")
        base = wid * b_per_w
        my_tab = tab_hbm.at[wid]

        @pl.loop(0, n_outer)
        def _(o):
            obase = base + o * IB
            pltpu.sync_copy(idx_hbm.at[pl.ds(obase, IB)], idx_v)

            @pl.loop(0, CPB, step=NB)
            def _(ci):
                handles = []
                for b in range(NB):
                    c = ci + b
                    off = obase + c * W

                    @pl.when(jnp.logical_or(o > 0, ci >= NB))
                    def _():
                        pltpu.make_async_copy(
                            rows[b], out_hbm.at[pl.ds(off, W)], ws[b]
                        ).wait()

                    handles.append(
                        pltpu.async_copy(
                            my_tab.at[idx_v.at[pl.ds(c * W, W)]],
                            rows[b],
                            gs[b],
                        )
                    )
                for b in range(NB):
                    c = ci + b
                    off = obase + c * W
                    handles[b].wait()
                    pltpu.async_copy(rows[b], out_hbm.at[pl.ds(off, W)], ws[b])

        for b in range(NB):
            pltpu.make_async_copy(
                rows[b], out_hbm.at[pl.ds(base, W)], ws[b]
            ).wait()

    return k(table_repl, tokens_flat)


def kernel(tokens, table):
    bsz, seq = tokens.shape
    table_repl = jnp.broadcast_to(table, (NW,) + table.shape)
    out = _sc_gather(tokens.reshape(bsz * seq), table_repl)
    return out.reshape(bsz, seq, D)


# speedup vs baseline: 5.7864x; 1.3015x over previous
import functools

import jax
import jax.numpy as jnp
from jax import lax
from jax.experimental import pallas as pl
from jax.experimental.pallas import tpu as pltpu
from jax.experimental.pallas import tpu_sc as plsc

D = 256
NC, NS = 2, 16
NW = NC * NS
W = 64
NB = 4
IB = 2048
CPB = IB // W


def _sc_gather(tokens_flat, table_repl):
    B = tokens_flat.shape[0]
    b_per_w = B // NW
    n_outer = b_per_w // IB
    mesh = plsc.VectorSubcoreMesh(core_axis_name="c", subcore_axis_name="s")

    @functools.partial(
        pl.kernel,
        mesh=mesh,
        out_type=jax.ShapeDtypeStruct((B, D), jnp.float32),
        scratch_types=(
            [pltpu.VMEM((IB,), jnp.int32)]
            + [pltpu.VMEM((W, D), jnp.float32)] * NB
            + [pltpu.SemaphoreType.DMA] * (2 * NB)
        ),
    )
    def k(tab_hbm, idx_hbm, out_hbm, idx_v, *bufs):
        rows = bufs[:NB]
        gs = bufs[NB:2 * NB]
        ws = bufs[2 * NB:3 * NB]
        wid = lax.axis_index("s") * NC + lax.axis_index("c")
        base = wid * b_per_w

        @pl.loop(0, n_outer)
        def _(o):
            obase = base + o * IB
            pltpu.sync_copy(idx_hbm.at[pl.ds(obase, IB)], idx_v)

            @pl.loop(0, CPB, step=NB)
            def _(ci):
                handles = []
                for b in range(NB):
                    c = ci + b
                    off = obase + c * W

                    @pl.when(jnp.logical_or(o > 0, ci >= NB))
                    def _():
                        pltpu.make_async_copy(
                            rows[b], out_hbm.at[pl.ds(off, W)], ws[b]
                        ).wait()

                    handles.append(
                        pltpu.async_copy(
                            tab_hbm.at[wid * 4 + b].at[
                                idx_v.at[pl.ds(c * W, W)]],
                            rows[b],
                            gs[b],
                        )
                    )
                for b in range(NB):
                    c = ci + b
                    off = obase + c * W
                    handles[b].wait()
                    pltpu.async_copy(rows[b], out_hbm.at[pl.ds(off, W)], ws[b])

        for b in range(NB):
            pltpu.make_async_copy(
                rows[b], out_hbm.at[pl.ds(base, W)], ws[b]
            ).wait()

    return k(table_repl, tokens_flat)


def kernel(tokens, table):
    bsz, seq = tokens.shape
    table_repl = jnp.broadcast_to(table, (NW * 4,) + table.shape)
    out = _sc_gather(tokens.reshape(bsz * seq), table_repl)
    return out.reshape(bsz, seq, D)


# speedup vs baseline: 6.1112x; 1.0561x over previous
import functools

import jax
import jax.numpy as jnp
from jax import lax
from jax.experimental import pallas as pl
from jax.experimental.pallas import tpu as pltpu
from jax.experimental.pallas import tpu_sc as plsc

D = 256
NC, NS = 2, 16
NW = NC * NS
W = 64
NB = 4
IB = 2048
CPB = IB // W


def _sc_gather(tokens_flat, table_repl):
    B = tokens_flat.shape[0]
    b_per_w = B // NW
    n_outer = b_per_w // IB
    mesh = plsc.VectorSubcoreMesh(core_axis_name="c", subcore_axis_name="s")

    @functools.partial(
        pl.kernel,
        mesh=mesh,
        out_type=jax.ShapeDtypeStruct((B, D), jnp.float32),
        scratch_types=(
            [pltpu.VMEM((IB,), jnp.int32)]
            + [pltpu.VMEM((W, D), jnp.float32)] * NB
            + [pltpu.SemaphoreType.DMA] * (2 * NB)
        ),
    )
    def k(tab_hbm, idx_hbm, out_hbm, idx_v, *bufs):
        rows = bufs[:NB]
        gs = bufs[NB:2 * NB]
        ws = bufs[2 * NB:3 * NB]
        wid = lax.axis_index("s") * NC + lax.axis_index("c")
        base = wid * b_per_w

        @pl.loop(0, n_outer)
        def _(o):
            obase = base + o * IB
            pltpu.sync_copy(idx_hbm.at[pl.ds(obase, IB)], idx_v)

            @pl.loop(0, CPB, step=NB)
            def _(ci):
                handles = []
                for b in range(NB):
                    c = ci + b
                    off = obase + c * W

                    @pl.when(jnp.logical_or(o > 0, ci >= NB))
                    def _():
                        pltpu.make_async_copy(
                            rows[b], out_hbm.at[pl.ds(off, W)], ws[b]
                        ).wait()

                    handles.append(
                        pltpu.async_copy(
                            tab_hbm.at[wid * 16 + b * 4 + ((ci // NB) & 3)].at[
                                idx_v.at[pl.ds(c * W, W)]],
                            rows[b],
                            gs[b],
                        )
                    )
                for b in range(NB):
                    c = ci + b
                    off = obase + c * W
                    handles[b].wait()
                    pltpu.async_copy(rows[b], out_hbm.at[pl.ds(off, W)], ws[b])

        for b in range(NB):
            pltpu.make_async_copy(
                rows[b], out_hbm.at[pl.ds(base, W)], ws[b]
            ).wait()

    return k(table_repl, tokens_flat)


def kernel(tokens, table):
    bsz, seq = tokens.shape
    table_repl = jnp.broadcast_to(table, (NW * 16,) + table.shape)
    out = _sc_gather(tokens.reshape(bsz * seq), table_repl)
    return out.reshape(bsz, seq, D)


# speedup vs baseline: 6.1562x; 1.0074x over previous
import functools

import jax
import jax.numpy as jnp
from jax import lax
from jax.experimental import pallas as pl
from jax.experimental.pallas import tpu as pltpu
from jax.experimental.pallas import tpu_sc as plsc

D = 256
NC, NS = 2, 16
NW = NC * NS
W = 64
NB = 4
R = 16
IB = 2048
CPB = IB // W


def _sc_gather(tokens_flat, table_repl):
    B = tokens_flat.shape[0]
    b_per_w = B // NW
    n_outer = b_per_w // IB
    mesh = plsc.VectorSubcoreMesh(core_axis_name="c", subcore_axis_name="s")

    @functools.partial(
        pl.kernel,
        mesh=mesh,
        out_type=jax.ShapeDtypeStruct((B, D), jnp.float32),
        scratch_types=(
            [pltpu.VMEM((IB,), jnp.int32)] * 2
            + [pltpu.VMEM((W, D), jnp.float32)] * NB
            + [pltpu.SemaphoreType.DMA] * (2 + 2 * NB)
        ),
    )
    def k(tab_hbm, idx_hbm, out_hbm, idx0, idx1, *bufs):
        rows = bufs[:NB]
        isem = bufs[NB:NB + 2]
        gs = bufs[NB + 2:NB + 2 + NB]
        ws = bufs[NB + 2 + NB:]
        idxs = (idx0, idx1)
        wid = lax.axis_index("s") * NC + lax.axis_index("c")
        base = wid * b_per_w

        def load_idx(o, p):
            pltpu.async_copy(
                idx_hbm.at[pl.ds(base + o * IB, IB)], idxs[p], isem[p]
            )

        def wait_idx(p):
            pltpu.make_async_copy(
                idx_hbm.at[pl.ds(base, IB)], idxs[p], isem[p]
            ).wait()

        def inner(o, idx_v):
            obase = base + o * IB

            @pl.loop(0, CPB, step=NB)
            def _(ci):
                handles = []
                for b in range(NB):
                    c = ci + b
                    off = obase + c * W

                    @pl.when(jnp.logical_or(o > 0, ci >= NB))
                    def _():
                        pltpu.make_async_copy(
                            rows[b], out_hbm.at[pl.ds(off, W)], ws[b]
                        ).wait()

                    handles.append(
                        pltpu.async_copy(
                            tab_hbm.at[wid * R + b * 4 + ((ci // NB) & 3)].at[
                                idx_v.at[pl.ds(c * W, W)]],
                            rows[b],
                            gs[b],
                        )
                    )
                for b in range(NB):
                    c = ci + b
                    off = obase + c * W
                    handles[b].wait()
                    pltpu.async_copy(rows[b], out_hbm.at[pl.ds(off, W)], ws[b])

        load_idx(0, 0)

        @pl.loop(0, n_outer, step=2)
        def _(o):
            load_idx(o + 1, 1)
            wait_idx(0)
            inner(o, idx0)

            @pl.when(o + 2 < n_outer)
            def _():
                load_idx(o + 2, 0)

            wait_idx(1)
            inner(o + 1, idx1)

        for b in range(NB):
            pltpu.make_async_copy(
                rows[b], out_hbm.at[pl.ds(base, W)], ws[b]
            ).wait()

    return k(table_repl, tokens_flat)


def kernel(tokens, table):
    bsz, seq = tokens.shape
    table_repl = jnp.broadcast_to(table, (NW * R,) + table.shape)
    out = _sc_gather(tokens.reshape(bsz * seq), table_repl)
    return out.reshape(bsz, seq, D)
